# baseline (speedup 1.0000x reference)
import jax
import jax.numpy as jnp
from jax import lax
from jax.experimental import pallas as pl
from jax.experimental.pallas import tpu as pltpu
from jax.experimental.pallas import tpu_sc as plsc

D_MODEL = 64
NUM_CORES = 2
NUM_SUBCORES = 16
NUM_WORKERS = NUM_CORES * NUM_SUBCORES
SEQ_PAD = 64
SLAB = 4
NSLAB = 4


def _gather_body(ids_hbm, table_hbm, out_hbm, idx_all, slabs, gsems, osems):
    wid = lax.axis_index("s") * NUM_CORES + lax.axis_index("c")
    seq = out_hbm.shape[1]
    rows_per_w = ids_hbm.shape[0] // NUM_WORKERS
    nslabs = rows_per_w // SLAB
    base = wid * rows_per_w

    pltpu.sync_copy(ids_hbm.at[pl.ds(base, rows_per_w)], idx_all)

    def start_slab(b, k):
        for j in range(SLAB):
            pltpu.async_copy(
                table_hbm.at[idx_all.at[k * SLAB + j]],
                slabs.at[b, j],
                gsems.at[b],
            )

    def wait_slab(b, k):
        for j in range(SLAB):
            pltpu.make_async_copy(
                table_hbm.at[idx_all.at[k * SLAB + j]],
                slabs.at[b, j],
                gsems.at[b],
            ).wait()

    def store_slab(b, k):
        for j in range(SLAB):
            pltpu.async_copy(
                slabs.at[b, j, pl.ds(0, seq)],
                out_hbm.at[base + k * SLAB + j],
                osems.at[b],
            )

    def wait_store(b, k):
        for j in range(SLAB):
            pltpu.make_async_copy(
                slabs.at[b, j, pl.ds(0, seq)],
                out_hbm.at[base + k * SLAB + j],
                osems.at[b],
            ).wait()

    for b in range(NSLAB):
        start_slab(b, b)

    @pl.loop(0, nslabs, step=NSLAB)
    def _(k0):
        for b in range(NSLAB):
            k = k0 + b
            wait_slab(b, k)
            store_slab(b, k)

            @pl.when(k + NSLAB < nslabs)
            def _():
                wait_store(b, k)
                start_slab(b, k + NSLAB)

    for b in range(NSLAB):
        wait_store(b, nslabs - NSLAB + b)


def kernel(token_ids, weight):
    n_tok, seq = token_ids.shape
    rows_per_w = n_tok // NUM_WORKERS
    ids_pad = jnp.pad(token_ids.astype(jnp.int32), ((0, 0), (0, SEQ_PAD - seq)))

    mesh = plsc.VectorSubcoreMesh(core_axis_name="c", subcore_axis_name="s")
    out = pl.kernel(
        _gather_body,
        out_type=jax.ShapeDtypeStruct((n_tok, seq, D_MODEL), jnp.float32),
        mesh=mesh,
        scratch_types=[
            pltpu.VMEM((rows_per_w, SEQ_PAD), jnp.int32),
            pltpu.VMEM((NSLAB, SLAB, SEQ_PAD, D_MODEL), jnp.float32),
            pltpu.SemaphoreType.DMA((NSLAB,)),
            pltpu.SemaphoreType.DMA((NSLAB,)),
        ],
        compiler_params=pltpu.CompilerParams(use_tc_tiling_on_sc=False),
    )(ids_pad, weight)
    return out


# speedup vs baseline: 4.5608x; 4.5608x over previous
import jax
import jax.numpy as jnp
from jax import lax
from jax.experimental import pallas as pl
from jax.experimental.pallas import tpu as pltpu
from jax.experimental.pallas import tpu_sc as plsc

D_MODEL = 64
NUM_CORES = 2
NUM_SUBCORES = 16
NUM_WORKERS = NUM_CORES * NUM_SUBCORES
CHUNK = 128
NBUF = 5
NSLICE = 4


def _gather_body(ids_hbm, table_hbm, out_hbm, idx_all, rows_v, gsems, osems):
    wid = lax.axis_index("s") * NUM_CORES + lax.axis_index("c")
    b_per_w = idx_all.shape[0]
    nchunks = b_per_w // CHUNK
    base = wid * b_per_w

    pltpu.sync_copy(ids_hbm.at[pl.ds(pl.multiple_of(base, 8), b_per_w)], idx_all)

    def start_gather(b, g):
        idx = idx_all.at[pl.ds(g * CHUNK, CHUNK)]
        pltpu.async_copy(table_hbm.at[idx], rows_v.at[b], gsems.at[b])

    for b in range(NBUF):
        start_gather(b, b)

    @pl.loop(0, nchunks, step=NBUF)
    def _(g0):
        for b in range(NBUF):
            g = g0 + b
            idx = idx_all.at[pl.ds(g * CHUNK, CHUNK)]
            pltpu.make_async_copy(
                table_hbm.at[idx], rows_v.at[b], gsems.at[b]
            ).wait()
            off = pl.multiple_of(base + g * CHUNK, 8)
            out_slice = out_hbm.at[pl.ds(off, CHUNK)]
            pltpu.async_copy(rows_v.at[b], out_slice, osems.at[b])

            @pl.when(g + NBUF < nchunks)
            def _():
                pltpu.make_async_copy(rows_v.at[b], out_slice, osems.at[b]).wait()
                start_gather(b, g + NBUF)

    for b in range(NBUF):
        g = nchunks - NBUF + b
        off = pl.multiple_of(base + g * CHUNK, 8)
        pltpu.make_async_copy(
            rows_v.at[b], out_hbm.at[pl.ds(off, CHUNK)], osems.at[b]
        ).wait()


def kernel(token_ids, weight):
    n_tok, seq = token_ids.shape
    b_total = n_tok * seq
    b_slice = b_total // NSLICE
    b_per_w = b_slice // NUM_WORKERS
    tok_slice = n_tok // NSLICE

    mesh = plsc.VectorSubcoreMesh(core_axis_name="c", subcore_axis_name="s")
    call = pl.kernel(
        _gather_body,
        out_type=jax.ShapeDtypeStruct((b_slice, D_MODEL), jnp.float32),
        mesh=mesh,
        scratch_types=[
            pltpu.VMEM((b_per_w,), jnp.int32),
            pltpu.VMEM((NBUF, CHUNK, D_MODEL), jnp.float32),
            pltpu.SemaphoreType.DMA((NBUF,)),
            pltpu.SemaphoreType.DMA((NBUF,)),
        ],
        compiler_params=pltpu.CompilerParams(use_tc_tiling_on_sc=False),
    )

    outs = []
    for i in range(NSLICE):
        ids_i = token_ids[i * tok_slice:(i + 1) * tok_slice]
        flat_i = ids_i.reshape(b_slice).astype(jnp.int32)
        out_i = call(flat_i, weight)
        outs.append(out_i.reshape(tok_slice, seq, D_MODEL))
    return jnp.concatenate(outs, axis=0)
